# baseline (speedup 1.0000x reference)
import functools
import jax
import jax.numpy as jnp
from jax import lax
from jax.experimental import pallas as pl
from jax.experimental.pallas import tpu as pltpu
from jax.experimental.pallas import tpu_sc as plsc

_B, _S, _D = 4, 8192, 1024
_NW = 32
_ROWS_PER_W = _S // _NW
_C = 4
_NCHUNK = _ROWS_PER_W // _C
_NBUF = 6
_K = 4
_LANES = 16
_GPR = _D // _LANES


def _sc_body(x_hbm, emb_hbm, out_hbm, emb_v, x_v, *sems):
    in_sems = sems[:_NBUF]
    out_sems = sems[_NBUF:]
    wid = lax.axis_index("s") * 2 + lax.axis_index("c")
    row0 = wid * _ROWS_PER_W

    def in_copies(c, u):
        r = row0 + c * _C
        return (
            pltpu.make_async_copy(
                emb_hbm.at[pl.ds(r, _C)], emb_v.at[u], in_sems[u]),
            pltpu.make_async_copy(
                x_hbm.at[:, pl.ds(r, _C)], x_v.at[u], in_sems[u]),
        )

    def out_copy(c, u):
        r = row0 + c * _C
        return pltpu.make_async_copy(
            x_v.at[u], out_hbm.at[:, pl.ds(r, _C)], out_sems[u])

    def compute(u):
        for row in range(_C):
            def col_body(k, c2, row=row):
                for v in range(4):
                    col = (k * 4 + v) * _LANES
                    e = emb_v[u, row, pl.ds(col, _LANES)]
                    for b in range(_B):
                        plsc.addupdate(
                            x_v.at[u, b, row, pl.ds(col, _LANES)], e)
                return c2
            lax.fori_loop(0, _GPR // 4, col_body, 0)

    for c in range(_K):
        for cp in in_copies(c, c % _NBUF):
            cp.start()

    _MAIN = _NCHUNK - _K

    def outer(i, carry):
        c0 = i * _NBUF
        for u in range(_NBUF):
            c = c0 + u
            uk = (u + _K) % _NBUF

            @pl.when(c >= _NBUF - _K)
            def _():
                out_copy(c - (_NBUF - _K), uk).wait()

            for cp in in_copies(c + _K, uk):
                cp.start()
            for cp in in_copies(c, u):
                cp.wait()
            compute(u)
            out_copy(c, u).start()
        return carry

    lax.fori_loop(0, _MAIN // _NBUF, outer, 0)
    for c in range(_MAIN, _NCHUNK):
        u = c % _NBUF
        for cp in in_copies(c, u):
            cp.wait()
        compute(u)
        out_copy(c, u).start()
    for c in range(_MAIN - 2, _NCHUNK):
        out_copy(c, c % _NBUF).wait()


def kernel(x, embeddings):
    mesh = plsc.VectorSubcoreMesh(core_axis_name="c", subcore_axis_name="s")
    run = functools.partial(
        pl.kernel,
        mesh=mesh,
        out_type=jax.ShapeDtypeStruct((_B, _S, _D), jnp.float32),
        scratch_types=[
            pltpu.VMEM((_NBUF, _C, _D), jnp.float32),
            pltpu.VMEM((_NBUF, _B, _C, _D), jnp.float32),
        ] + [pltpu.SemaphoreType.DMA] * (2 * _NBUF),
    )(_sc_body)
    return run(x, embeddings)


# speedup vs baseline: 1.0437x; 1.0437x over previous
import functools
import jax
import jax.numpy as jnp
from jax import lax
from jax.experimental import pallas as pl
from jax.experimental.pallas import tpu as pltpu
from jax.experimental.pallas import tpu_sc as plsc

_B, _S, _D = 4, 8192, 1024
_NW = 32
_ROWS_PER_W = _S // _NW
_C = 2
_NCHUNK = _ROWS_PER_W // _C
_NBUF = 8
_K = 6
_LANES = 16
_GPR = _D // _LANES


def _sc_body(x_hbm, emb_hbm, out_hbm, emb_v, x_v, *sems):
    in_sems = sems[:_NBUF]
    out_sems = sems[_NBUF:]
    wid = lax.axis_index("s") * 2 + lax.axis_index("c")
    row0 = wid * _ROWS_PER_W

    def in_copies(c, u):
        r = row0 + c * _C
        return (
            pltpu.make_async_copy(
                emb_hbm.at[pl.ds(r, _C)], emb_v.at[u], in_sems[u]),
            pltpu.make_async_copy(
                x_hbm.at[:, pl.ds(r, _C)], x_v.at[u], in_sems[u]),
        )

    def out_copy(c, u):
        r = row0 + c * _C
        return pltpu.make_async_copy(
            x_v.at[u], out_hbm.at[:, pl.ds(r, _C)], out_sems[u])

    def compute(u):
        for row in range(_C):
            def col_body(k, c2, row=row):
                for v in range(4):
                    col = (k * 4 + v) * _LANES
                    e = emb_v[u, row, pl.ds(col, _LANES)]
                    for b in range(_B):
                        plsc.addupdate(
                            x_v.at[u, b, row, pl.ds(col, _LANES)], e)
                return c2
            lax.fori_loop(0, _GPR // 4, col_body, 0)

    for c in range(_K):
        for cp in in_copies(c, c % _NBUF):
            cp.start()

    def outer(i, carry):
        c0 = i * _NBUF
        for u in range(_NBUF):
            c = c0 + u
            uk = (u + _K) % _NBUF

            @pl.when(c >= _NBUF - _K)
            def _():
                out_copy(c - (_NBUF - _K), uk).wait()

            @pl.when(c + _K < _NCHUNK)
            def _():
                for cp in in_copies(c + _K, uk):
                    cp.start()

            for cp in in_copies(c, u):
                cp.wait()
            compute(u)
            out_copy(c, u).start()
        return carry

    lax.fori_loop(0, _NCHUNK // _NBUF, outer, 0)
    for c in range(_NCHUNK - (_NBUF - _K), _NCHUNK):
        out_copy(c, c % _NBUF).wait()


def kernel(x, embeddings):
    mesh = plsc.VectorSubcoreMesh(core_axis_name="c", subcore_axis_name="s")
    run = functools.partial(
        pl.kernel,
        mesh=mesh,
        out_type=jax.ShapeDtypeStruct((_B, _S, _D), jnp.float32),
        scratch_types=[
            pltpu.VMEM((_NBUF, _C, _D), jnp.float32),
            pltpu.VMEM((_NBUF, _B, _C, _D), jnp.float32),
        ] + [pltpu.SemaphoreType.DMA] * (2 * _NBUF),
    )(_sc_body)
    return run(x, embeddings)
